# baseline (speedup 1.0000x reference)
import functools

import jax
import jax.numpy as jnp
from jax import lax
from jax.experimental import pallas as pl
from jax.experimental.pallas import tpu as pltpu
from jax.experimental.pallas import tpu_sc as plsc

_NC = 2
_NS = 16
_NW = _NC * _NS


def _copy_body(layer_elems, per_tile, chunks_per_layer, in_hbm, out_hbm, sem):
    wid = lax.axis_index("s") * _NC + lax.axis_index("c")
    layer = wid // chunks_per_layer
    chunk = wid % chunks_per_layer
    in_off = (2 * layer + 1) * layer_elems + chunk * per_tile
    out_off = wid * per_tile
    pltpu.async_copy(
        in_hbm.at[pl.ds(in_off, per_tile)],
        out_hbm.at[pl.ds(out_off, per_tile)],
        sem,
    ).wait()


def kernel(hidden_states):
    num_dirs_layers, batch, hidden = hidden_states.shape
    num_layers = num_dirs_layers // 2
    layer_elems = batch * hidden
    out_elems = num_layers * layer_elems
    per_tile = out_elems // _NW
    chunks_per_layer = layer_elems // per_tile

    flat_in = hidden_states.reshape(-1)
    mesh = plsc.VectorSubcoreMesh(core_axis_name="c", subcore_axis_name="s")
    out_flat = pl.kernel(
        functools.partial(_copy_body, layer_elems, per_tile, chunks_per_layer),
        mesh=mesh,
        out_type=jax.ShapeDtypeStruct((out_elems,), jnp.float32),
        scratch_types=[pltpu.SemaphoreType.DMA],
    )(flat_in)
    return out_flat.reshape(num_layers, batch, hidden)


# speedup vs baseline: 1.0745x; 1.0745x over previous
import functools

import jax
import jax.numpy as jnp
from jax import lax
from jax.experimental import pallas as pl
from jax.experimental.pallas import tpu as pltpu
from jax.experimental.pallas import tpu_sc as plsc

_NC = 2
_NS = 16
_NW = _NC * _NS


def _copy_body(rows_per_tile, chunks_per_layer, in_hbm, out_hbm, sem):
    wid = lax.axis_index("s") * _NC + lax.axis_index("c")
    layer = wid // chunks_per_layer
    row0 = (wid % chunks_per_layer) * rows_per_tile
    pltpu.async_copy(
        in_hbm.at[2 * layer + 1, pl.ds(row0, rows_per_tile)],
        out_hbm.at[layer, pl.ds(row0, rows_per_tile)],
        sem,
    ).wait()


def kernel(hidden_states):
    num_dirs_layers, batch, hidden = hidden_states.shape
    num_layers = num_dirs_layers // 2
    chunks_per_layer = _NW // num_layers
    rows_per_tile = batch // chunks_per_layer

    mesh = plsc.VectorSubcoreMesh(core_axis_name="c", subcore_axis_name="s")
    return pl.kernel(
        functools.partial(_copy_body, rows_per_tile, chunks_per_layer),
        mesh=mesh,
        out_type=jax.ShapeDtypeStruct((num_layers, batch, hidden), jnp.float32),
        scratch_types=[pltpu.SemaphoreType.DMA],
    )(hidden_states)


# speedup vs baseline: 11.8655x; 11.0428x over previous
import functools

import jax
import jax.numpy as jnp
from jax import lax
from jax.experimental import pallas as pl
from jax.experimental.pallas import tpu as pltpu
from jax.experimental.pallas import tpu_sc as plsc

_NC = 2
_NS = 16
_NW = _NC * _NS


def _copy_body(rows_per_tile, chunks_per_layer, in_hbm, out_hbm, buf):
    wid = lax.axis_index("s") * _NC + lax.axis_index("c")
    layer = wid // chunks_per_layer
    row0 = (wid % chunks_per_layer) * rows_per_tile
    pltpu.sync_copy(in_hbm.at[2 * layer + 1, pl.ds(row0, rows_per_tile)], buf)
    pltpu.sync_copy(buf, out_hbm.at[layer, pl.ds(row0, rows_per_tile)])


def kernel(hidden_states):
    num_dirs_layers, batch, hidden = hidden_states.shape
    num_layers = num_dirs_layers // 2
    chunks_per_layer = _NW // num_layers
    rows_per_tile = batch // chunks_per_layer

    mesh = plsc.VectorSubcoreMesh(core_axis_name="c", subcore_axis_name="s")
    return pl.kernel(
        functools.partial(_copy_body, rows_per_tile, chunks_per_layer),
        mesh=mesh,
        out_type=jax.ShapeDtypeStruct((num_layers, batch, hidden), jnp.float32),
        scratch_types=[pltpu.VMEM((rows_per_tile, hidden), jnp.float32)],
    )(hidden_states)
